# baseline (speedup 1.0000x reference)
import jax
import jax.numpy as jnp
from jax import lax
from jax.experimental import pallas as pl
from jax.experimental.pallas import tpu as pltpu
from jax.experimental.pallas import tpu_sc as plsc

_D = 768
_E = 64
_H = 512
_CAP = 64
_N = 2048
_SENTINEL = _E * _CAP
_SLOTS_PAD = _E * _CAP + 128
_NW = 32
_TPW = _N // _NW
_TPS = _N // 16


def _routing_kernel(xf_ref, wr_ref, slot_ref, scale_ref):
    xf = xf_ref[...]
    wr = wr_ref[...]
    logits = jnp.dot(xf, wr, preferred_element_type=jnp.float32)
    m = jnp.max(logits, axis=1, keepdims=True)
    w0 = 1.0 / jnp.sum(jnp.exp(logits - m), axis=1, keepdims=True)
    lane = jax.lax.broadcasted_iota(jnp.int32, (_N, _E), 1)
    cand = jnp.where(logits >= m, lane, _E)
    idx0 = jnp.min(cand, axis=1, keepdims=True)
    onehot = (lane == idx0).astype(jnp.float32)
    r = jax.lax.broadcasted_iota(jnp.int32, (_N, _N), 0)
    c = jax.lax.broadcasted_iota(jnp.int32, (_N, _N), 1)
    tri = (r >= c).astype(jnp.float32)
    pos = jnp.dot(tri, onehot, preferred_element_type=jnp.float32)
    pos_in = jnp.sum(pos * onehot, axis=1, keepdims=True) - 1.0
    keep = pos_in < _CAP
    pos_c = jnp.clip(pos_in.astype(jnp.int32), 0, _CAP - 1)
    slot = idx0 * _CAP + pos_c
    slot_ref[...] = jnp.where(keep, slot, _SENTINEL)
    scale_ref[...] = jnp.where(keep, w0, 0.0)


def _inv_body(slot_hbm, scale_hbm, inv_hbm, ssc_hbm,
              idx_v, val_v, scl_v, buf_v, zf_v, sh_inv, sh_scl, sem):
    cid = lax.axis_index("c")
    sid = lax.axis_index("s")

    @pl.when(cid == 0)
    def _():
        base = sid * _TPS
        pltpu.sync_copy(slot_hbm.at[pl.ds(base, _TPS)], idx_v)
        pltpu.sync_copy(scale_hbm.at[pl.ds(base, _TPS)], scl_v)
        for j in range(_TPS // 16):
            val_v[pl.ds(j * 16, 16)] = (
                lax.iota(jnp.int32, 16) + (base + j * 16 + 1))

        @pl.when(sid == 0)
        def _():
            def zb(i, carry):
                buf_v[pl.ds(i * 16, 16)] = jnp.zeros((16,), jnp.int32)
                zf_v[pl.ds(i * 16, 16)] = jnp.zeros((16,), jnp.float32)
                return carry
            lax.fori_loop(0, _SLOTS_PAD // 16, zb, 0)
            pltpu.sync_copy(buf_v, sh_inv)
            pltpu.sync_copy(zf_v, sh_scl)

        plsc.subcore_barrier()
        pltpu.sync_copy(val_v, sh_inv.at[idx_v], add=True)
        pltpu.sync_copy(scl_v, sh_scl.at[idx_v], add=True)
        plsc.subcore_barrier()

        @pl.when(sid == 0)
        def _():
            pltpu.sync_copy(sh_inv, inv_hbm)
            pltpu.sync_copy(sh_scl, ssc_hbm)


_inv_call = pl.kernel(
    _inv_body,
    out_type=(
        jax.ShapeDtypeStruct((_SLOTS_PAD,), jnp.int32),
        jax.ShapeDtypeStruct((_SLOTS_PAD,), jnp.float32),
    ),
    mesh=plsc.VectorSubcoreMesh(core_axis_name="c", subcore_axis_name="s"),
    scratch_types=[
        pltpu.VMEM((_TPS,), jnp.int32),
        pltpu.VMEM((_TPS,), jnp.int32),
        pltpu.VMEM((_TPS,), jnp.float32),
        pltpu.VMEM((_SLOTS_PAD,), jnp.int32),
        pltpu.VMEM((_SLOTS_PAD,), jnp.float32),
        pltpu.VMEM_SHARED((_SLOTS_PAD,), jnp.int32),
        pltpu.VMEM_SHARED((_SLOTS_PAD,), jnp.float32),
        pltpu.SemaphoreType.DMA,
    ],
)


def _ffn_fused(inv_ref, xf_ref, ssc_ref, wi_ref, wo_ref, out_ref, disp_scr):
    e = pl.program_id(0)

    @pl.when(e == 0)
    def _():
        out_ref[...] = jnp.zeros((_N + 16, _D), jnp.float32)

    for c in range(_CAP):
        v = inv_ref[e * _CAP + c]
        t = jnp.maximum(v - 1, 0)
        disp_scr[pl.ds(c, 1), :] = xf_ref[pl.ds(t, 1), :]

    h = jnp.dot(disp_scr[...], wi_ref[0], preferred_element_type=jnp.float32)
    h = h * (1.0 / (1.0 + jnp.exp(-h)))
    eo = jnp.dot(h, wo_ref[0], preferred_element_type=jnp.float32)
    eo = eo * ssc_ref[...]

    for c in range(_CAP):
        v = inv_ref[e * _CAP + c]
        t = jnp.where(v > 0, v - 1, _N)
        out_ref[pl.ds(t, 1), :] = eo[c:c + 1, :]


def kernel(x, W_router, W_in, W_out):
    xf = x.reshape(_N, _D)
    slot, scale = pl.pallas_call(
        _routing_kernel,
        out_shape=(
            jax.ShapeDtypeStruct((_N, 1), jnp.int32),
            jax.ShapeDtypeStruct((_N, 1), jnp.float32),
        ),
    )(xf, W_router)

    inv, ssc = _inv_call(slot.reshape(_N), scale.reshape(_N))

    grid_spec = pltpu.PrefetchScalarGridSpec(
        num_scalar_prefetch=1,
        grid=(_E,),
        in_specs=[
            pl.BlockSpec((_N, _D), lambda e, inv_r: (0, 0)),
            pl.BlockSpec((_CAP, 1), lambda e, inv_r: (e, 0)),
            pl.BlockSpec((1, _D, _H), lambda e, inv_r: (e, 0, 0)),
            pl.BlockSpec((1, _H, _D), lambda e, inv_r: (e, 0, 0)),
        ],
        out_specs=pl.BlockSpec((_N + 16, _D), lambda e, inv_r: (0, 0)),
        scratch_shapes=[pltpu.VMEM((_CAP, _D), jnp.float32)],
    )
    out = pl.pallas_call(
        _ffn_fused,
        grid_spec=grid_spec,
        out_shape=jax.ShapeDtypeStruct((_N + 16, _D), jnp.float32),
    )(inv, xf, ssc.reshape(_SLOTS_PAD, 1), W_in, W_out)
    return out[:_N].reshape(x.shape)


# speedup vs baseline: 1.0076x; 1.0076x over previous
import jax
import jax.numpy as jnp
from jax import lax
from jax.experimental import pallas as pl
from jax.experimental.pallas import tpu as pltpu
from jax.experimental.pallas import tpu_sc as plsc

_D = 768
_E = 64
_H = 512
_CAP = 64
_N = 2048
_SENTINEL = _E * _CAP
_SLOTS_PAD = _E * _CAP + 128
_SLOTS_OUT = _E * _CAP + _CAP
_NW = 32
_TPW = _N // _NW
_TPS = _N // 16


def _routing_kernel(xf_ref, wr_ref, slot_ref, scale_ref):
    xf = xf_ref[...]
    wr = wr_ref[...]
    logits = jnp.dot(xf, wr, preferred_element_type=jnp.float32)
    m = jnp.max(logits, axis=1, keepdims=True)
    w0 = 1.0 / jnp.sum(jnp.exp(logits - m), axis=1, keepdims=True)
    lane = jax.lax.broadcasted_iota(jnp.int32, (_N, _E), 1)
    cand = jnp.where(logits >= m, lane, _E)
    idx0 = jnp.min(cand, axis=1, keepdims=True)
    onehot = (lane == idx0).astype(jnp.float32)
    r = jax.lax.broadcasted_iota(jnp.int32, (_N, _N), 0)
    c = jax.lax.broadcasted_iota(jnp.int32, (_N, _N), 1)
    tri = (r >= c).astype(jnp.float32)
    pos = jnp.dot(tri, onehot, preferred_element_type=jnp.float32)
    pos_in = jnp.sum(pos * onehot, axis=1, keepdims=True) - 1.0
    keep = pos_in < _CAP
    pos_c = jnp.clip(pos_in.astype(jnp.int32), 0, _CAP - 1)
    slot = idx0 * _CAP + pos_c
    slot_ref[...] = jnp.where(keep, slot, _SENTINEL)
    scale_ref[...] = jnp.where(keep, w0, 0.0)


def _inv_body(slot_hbm, scale_hbm, inv_hbm, ssc_hbm,
              idx_v, val_v, scl_v, buf_v, zf_v, sh_inv, sh_scl, sem):
    cid = lax.axis_index("c")
    sid = lax.axis_index("s")

    @pl.when(cid == 0)
    def _():
        base = sid * _TPS
        pltpu.sync_copy(slot_hbm.at[pl.ds(base, _TPS)], idx_v)
        pltpu.sync_copy(scale_hbm.at[pl.ds(base, _TPS)], scl_v)
        for j in range(_TPS // 16):
            val_v[pl.ds(j * 16, 16)] = (
                lax.iota(jnp.int32, 16) + (base + j * 16 + 1))

        @pl.when(sid == 0)
        def _():
            def zb(i, carry):
                buf_v[pl.ds(i * 16, 16)] = jnp.zeros((16,), jnp.int32)
                zf_v[pl.ds(i * 16, 16)] = jnp.zeros((16,), jnp.float32)
                return carry
            lax.fori_loop(0, _SLOTS_PAD // 16, zb, 0)
            pltpu.sync_copy(buf_v, sh_inv)
            pltpu.sync_copy(zf_v, sh_scl)

        plsc.subcore_barrier()
        pltpu.sync_copy(val_v, sh_inv.at[idx_v], add=True)
        pltpu.sync_copy(scl_v, sh_scl.at[idx_v], add=True)
        plsc.subcore_barrier()

        @pl.when(sid == 0)
        def _():
            pltpu.sync_copy(sh_inv, inv_hbm)
            pltpu.sync_copy(sh_scl, ssc_hbm)


_inv_call = pl.kernel(
    _inv_body,
    out_type=(
        jax.ShapeDtypeStruct((_SLOTS_PAD,), jnp.int32),
        jax.ShapeDtypeStruct((_SLOTS_PAD,), jnp.float32),
    ),
    mesh=plsc.VectorSubcoreMesh(core_axis_name="c", subcore_axis_name="s"),
    scratch_types=[
        pltpu.VMEM((_TPS,), jnp.int32),
        pltpu.VMEM((_TPS,), jnp.int32),
        pltpu.VMEM((_TPS,), jnp.float32),
        pltpu.VMEM((_SLOTS_PAD,), jnp.int32),
        pltpu.VMEM((_SLOTS_PAD,), jnp.float32),
        pltpu.VMEM_SHARED((_SLOTS_PAD,), jnp.int32),
        pltpu.VMEM_SHARED((_SLOTS_PAD,), jnp.float32),
        pltpu.SemaphoreType.DMA,
    ],
)


def _ffn_kernel(inv_ref, xf_ref, ssc_ref, wi_ref, wo_ref, eo_ref, disp_scr):
    e = pl.program_id(0)

    @pl.when(e < _E)
    def _():
        for c in range(_CAP):
            v = inv_ref[e * _CAP + c]
            t = jnp.maximum(v - 1, 0)
            disp_scr[pl.ds(c, 1), :] = xf_ref[pl.ds(t, 1), :]

        h = jnp.dot(disp_scr[...], wi_ref[0],
                    preferred_element_type=jnp.float32)
        h = h * (1.0 / (1.0 + jnp.exp(-h)))
        eo = jnp.dot(h, wo_ref[0], preferred_element_type=jnp.float32)
        eo_ref[...] = eo * ssc_ref[...]

    @pl.when(e == _E)
    def _():
        eo_ref[...] = jnp.zeros((_CAP, _D), jnp.float32)


def _comb_body(eo_hbm, slot_hbm, out_hbm, idx_v, rows_v, sem):
    wid = lax.axis_index("s") * 2 + lax.axis_index("c")
    base = wid * _TPW
    pltpu.sync_copy(slot_hbm.at[pl.ds(base, _TPW)], idx_v)
    pltpu.async_copy(eo_hbm.at[idx_v], rows_v, sem).wait()
    pltpu.sync_copy(rows_v, out_hbm.at[pl.ds(base, _TPW)])


_comb_call = pl.kernel(
    _comb_body,
    out_type=jax.ShapeDtypeStruct((_N, _D), jnp.float32),
    mesh=plsc.VectorSubcoreMesh(core_axis_name="c", subcore_axis_name="s"),
    scratch_types=[
        pltpu.VMEM((_TPW,), jnp.int32),
        pltpu.VMEM((_TPW, _D), jnp.float32),
        pltpu.SemaphoreType.DMA,
    ],
)


def kernel(x, W_router, W_in, W_out):
    xf = x.reshape(_N, _D)
    slot, scale = pl.pallas_call(
        _routing_kernel,
        out_shape=(
            jax.ShapeDtypeStruct((_N, 1), jnp.int32),
            jax.ShapeDtypeStruct((_N, 1), jnp.float32),
        ),
    )(xf, W_router)
    slot1 = slot.reshape(_N)

    inv, ssc = _inv_call(slot1, scale.reshape(_N))

    clamp3 = lambda e, inv_r: (jnp.minimum(e, _E - 1), 0, 0)
    grid_spec = pltpu.PrefetchScalarGridSpec(
        num_scalar_prefetch=1,
        grid=(_E + 1,),
        in_specs=[
            pl.BlockSpec((_N, _D), lambda e, inv_r: (0, 0)),
            pl.BlockSpec((_CAP, 1), lambda e, inv_r: (jnp.minimum(e, _E - 1), 0)),
            pl.BlockSpec((1, _D, _H), clamp3),
            pl.BlockSpec((1, _H, _D), clamp3),
        ],
        out_specs=pl.BlockSpec((_CAP, _D), lambda e, inv_r: (e, 0)),
        scratch_shapes=[pltpu.VMEM((_CAP, _D), jnp.float32)],
    )
    eo = pl.pallas_call(
        _ffn_kernel,
        grid_spec=grid_spec,
        out_shape=jax.ShapeDtypeStruct((_SLOTS_OUT, _D), jnp.float32),
    )(inv, xf, ssc.reshape(_SLOTS_PAD, 1), W_in, W_out)

    out = _comb_call(eo, slot1)
    return out.reshape(x.shape)


# speedup vs baseline: 1.0601x; 1.0521x over previous
import jax
import jax.numpy as jnp
from jax import lax
from jax.experimental import pallas as pl
from jax.experimental.pallas import tpu as pltpu
from jax.experimental.pallas import tpu_sc as plsc

_D = 768
_E = 64
_H = 512
_CAP = 64
_N = 2048
_SENTINEL = _E * _CAP
_SLOTS = _E * _CAP + _CAP
_NW = 32
_TPW = _N // _NW


def _routing_kernel(xf_ref, wr_ref, slot_ref, scale_ref):
    xf = xf_ref[...]
    wr = wr_ref[...]
    logits = jnp.dot(xf, wr, preferred_element_type=jnp.float32)
    m = jnp.max(logits, axis=1, keepdims=True)
    w0 = 1.0 / jnp.sum(jnp.exp(logits - m), axis=1, keepdims=True)
    lane = jax.lax.broadcasted_iota(jnp.int32, (_N, _E), 1)
    cand = jnp.where(logits >= m, lane, _E)
    idx0 = jnp.min(cand, axis=1, keepdims=True)
    onehot = (lane == idx0).astype(jnp.float32)
    r = jax.lax.broadcasted_iota(jnp.int32, (_N, _N), 0)
    c = jax.lax.broadcasted_iota(jnp.int32, (_N, _N), 1)
    tri = (r >= c).astype(jnp.float32)
    pos = jnp.dot(tri, onehot, preferred_element_type=jnp.float32)
    pos_in = jnp.sum(pos * onehot, axis=1, keepdims=True) - 1.0
    keep = pos_in < _CAP
    pos_c = jnp.clip(pos_in.astype(jnp.int32), 0, _CAP - 1)
    slot = idx0 * _CAP + pos_c
    slot_ref[...] = jnp.where(keep, slot, _SENTINEL)
    scale_ref[...] = jnp.broadcast_to(jnp.where(keep, w0, 0.0), (_N, 128))


def _disp_body(xf_hbm, slot_hbm, scale_hbm, disp_hbm, ssc_hbm,
               idx_v, rows_v, scl_v, sem1, sem2):
    wid = lax.axis_index("s") * 2 + lax.axis_index("c")
    base = wid * _TPW
    pltpu.sync_copy(slot_hbm.at[pl.ds(base, _TPW)], idx_v)
    pltpu.sync_copy(xf_hbm.at[pl.ds(base, _TPW)], rows_v)
    cp1 = pltpu.async_copy(rows_v, disp_hbm.at[idx_v], sem1)
    pltpu.sync_copy(scale_hbm.at[pl.ds(base, _TPW)], scl_v)
    cp2 = pltpu.async_copy(scl_v, ssc_hbm.at[idx_v], sem2)
    cp1.wait()
    cp2.wait()


def _ffn_kernel(disp_ref, ssc_ref, wi_ref, wo_ref, eo_ref):
    e = pl.program_id(0)

    @pl.when(e < _E)
    def _():
        h = jnp.dot(disp_ref[...], wi_ref[0], preferred_element_type=jnp.float32)
        h = h * (1.0 / (1.0 + jnp.exp(-h)))
        eo = jnp.dot(h, wo_ref[0], preferred_element_type=jnp.float32)
        eo_ref[...] = eo * ssc_ref[:, 0:1]

    @pl.when(e == _E)
    def _():
        eo_ref[...] = jnp.zeros((_CAP, _D), jnp.float32)


def _comb_body(eo_hbm, slot_hbm, out_hbm, idx_v, rows_v, sem):
    wid = lax.axis_index("s") * 2 + lax.axis_index("c")
    base = wid * _TPW
    pltpu.sync_copy(slot_hbm.at[pl.ds(base, _TPW)], idx_v)
    pltpu.async_copy(eo_hbm.at[idx_v], rows_v, sem).wait()
    pltpu.sync_copy(rows_v, out_hbm.at[pl.ds(base, _TPW)])


_sc_mesh = plsc.VectorSubcoreMesh(core_axis_name="c", subcore_axis_name="s")

_disp_call = pl.kernel(
    _disp_body,
    out_type=(
        jax.ShapeDtypeStruct((_SLOTS, _D), jnp.float32),
        jax.ShapeDtypeStruct((_SLOTS, 128), jnp.float32),
    ),
    mesh=_sc_mesh,
    scratch_types=[
        pltpu.VMEM((_TPW,), jnp.int32),
        pltpu.VMEM((_TPW, _D), jnp.float32),
        pltpu.VMEM((_TPW, 128), jnp.float32),
        pltpu.SemaphoreType.DMA,
        pltpu.SemaphoreType.DMA,
    ],
)

_comb_call = pl.kernel(
    _comb_body,
    out_type=jax.ShapeDtypeStruct((_N, _D), jnp.float32),
    mesh=_sc_mesh,
    scratch_types=[
        pltpu.VMEM((_TPW,), jnp.int32),
        pltpu.VMEM((_TPW, _D), jnp.float32),
        pltpu.SemaphoreType.DMA,
    ],
)


def kernel(x, W_router, W_in, W_out):
    xf = x.reshape(_N, _D)
    slot, scale = pl.pallas_call(
        _routing_kernel,
        out_shape=(
            jax.ShapeDtypeStruct((_N, 1), jnp.int32),
            jax.ShapeDtypeStruct((_N, 128), jnp.float32),
        ),
    )(xf, W_router)
    slot1 = slot.reshape(_N)

    disp, ssc = _disp_call(xf, slot1, scale)

    clamp = lambda e: (jnp.minimum(e, _E - 1), 0)
    eo = pl.pallas_call(
        _ffn_kernel,
        grid=(_E + 1,),
        in_specs=[
            pl.BlockSpec((_CAP, _D), clamp),
            pl.BlockSpec((_CAP, 128), clamp),
            pl.BlockSpec((1, _D, _H), lambda e: (jnp.minimum(e, _E - 1), 0, 0)),
            pl.BlockSpec((1, _H, _D), lambda e: (jnp.minimum(e, _E - 1), 0, 0)),
        ],
        out_specs=pl.BlockSpec((_CAP, _D), lambda e: (e, 0)),
        out_shape=jax.ShapeDtypeStruct((_SLOTS, _D), jnp.float32),
    )(disp, ssc, W_in, W_out)

    out = _comb_call(eo, slot1)
    return out.reshape(x.shape)
